# Initial kernel scaffold; baseline (speedup 1.0000x reference)
import functools

import jax
import jax.numpy as jnp
from jax import lax
from jax.experimental import pallas as pl
from jax.experimental.pallas import tpu as pltpu
from jax.experimental.pallas import tpu_sc as plsc

N = 10000
E = 320000
D = 128
NC = 2
NS = 16
EPW = E // (NC * NS)
C = 125
NCHUNK = EPW // C
RPT = N // NS
RCH = 5

_mesh = plsc.VectorSubcoreMesh(
    core_axis_name="c", subcore_axis_name="s", num_cores=NC, num_subcores=NS)


def _pool_body(h_hbm, src_hbm, dst_hbm, zer_hbm, zed_hbm, one_hbm,
               part_out, deg_out,
               src_v, dst_v, rows_v, ones_v, sem, agg_sh, deg_sh):
    c = lax.axis_index("c")
    s = lax.axis_index("s")

    def zbody(i, carry):
        base = s * RPT + i * C
        pltpu.sync_copy(zer_hbm, agg_sh.at[pl.ds(base, C)])
        pltpu.sync_copy(zed_hbm, deg_sh.at[pl.ds(base, C)])
        return carry
    lax.fori_loop(0, RCH, zbody, 0)

    pltpu.sync_copy(src_hbm.at[c, s], src_v)
    pltpu.sync_copy(dst_hbm.at[c, s], dst_v)
    pltpu.sync_copy(one_hbm, ones_v)
    plsc.subcore_barrier()

    def ebody(i, carry):
        pltpu.async_copy(h_hbm.at[src_v.at[i]], rows_v, sem).wait()
        pltpu.sync_copy(rows_v, agg_sh.at[dst_v.at[i]], add=True)
        pltpu.sync_copy(ones_v, deg_sh.at[dst_v.at[i]], add=True)
        return carry
    lax.fori_loop(0, NCHUNK, ebody, 0)
    plsc.subcore_barrier()

    def obody(i, carry):
        base = s * RPT + i * C
        pltpu.sync_copy(agg_sh.at[pl.ds(base, C)], rows_v)
        pltpu.sync_copy(rows_v, part_out.at[c, pl.ds(base, C)])
        pltpu.sync_copy(deg_sh.at[pl.ds(base, C)], ones_v)
        pltpu.sync_copy(ones_v, deg_out.at[c, pl.ds(base, C)])
        return carry
    lax.fori_loop(0, RCH, obody, 0)


_pool = pl.kernel(
    _pool_body,
    out_type=(jax.ShapeDtypeStruct((NC, N, D), jnp.float32),
              jax.ShapeDtypeStruct((NC, N, 16), jnp.float32)),
    mesh=_mesh,
    scratch_types=(
        pltpu.VMEM((NCHUNK, C), jnp.int32),
        pltpu.VMEM((NCHUNK, C), jnp.int32),
        pltpu.VMEM((C, D), jnp.float32),
        pltpu.VMEM((C, 16), jnp.float32),
        pltpu.SemaphoreType.DMA,
        pltpu.VMEM_SHARED((N, D), jnp.float32),
        pltpu.VMEM_SHARED((N, 16), jnp.float32),
    ),
)


RB = 1000


def _gemm1_body(x_ref, w_ref, b_ref, o_ref):
    o_ref[...] = (jnp.dot(x_ref[...], w_ref[...],
                          preferred_element_type=jnp.float32)
                  + b_ref[...][None, :])


_gemm1 = pl.pallas_call(
    _gemm1_body,
    grid=(N // RB,),
    in_specs=[
        pl.BlockSpec((RB, D), lambda i: (i, 0)),
        pl.BlockSpec((D, D), lambda i: (0, 0)),
        pl.BlockSpec((D,), lambda i: (0,)),
    ],
    out_specs=pl.BlockSpec((RB, D), lambda i: (i, 0)),
    out_shape=jax.ShapeDtypeStruct((N, D), jnp.float32),
)


def _norm(p_ref, d_ref):
    p = p_ref[0] + p_ref[1]
    deg = jnp.sum(d_ref[0] + d_ref[1], axis=1, keepdims=True) / 16.0
    return p / jnp.maximum(deg, 1.0)


def _comb_gemm_body(p_ref, d_ref, w_ref, b_ref, o_ref):
    pooled = _norm(p_ref, d_ref)
    o_ref[...] = (jnp.dot(pooled, w_ref[...],
                          preferred_element_type=jnp.float32)
                  + b_ref[...][None, :])


_comb_gemm = pl.pallas_call(
    _comb_gemm_body,
    grid=(N // RB,),
    in_specs=[
        pl.BlockSpec((NC, RB, D), lambda i: (0, i, 0)),
        pl.BlockSpec((NC, RB, 16), lambda i: (0, i, 0)),
        pl.BlockSpec((D, D), lambda i: (0, 0)),
        pl.BlockSpec((D,), lambda i: (0,)),
    ],
    out_specs=pl.BlockSpec((RB, D), lambda i: (i, 0)),
    out_shape=jax.ShapeDtypeStruct((N, D), jnp.float32),
)


def _comb_body(p_ref, d_ref, o_ref):
    o_ref[...] = _norm(p_ref, d_ref)


_comb = pl.pallas_call(
    _comb_body,
    grid=(N // RB,),
    in_specs=[
        pl.BlockSpec((NC, RB, D), lambda i: (0, i, 0)),
        pl.BlockSpec((NC, RB, 16), lambda i: (0, i, 0)),
    ],
    out_specs=pl.BlockSpec((RB, D), lambda i: (i, 0)),
    out_shape=jax.ShapeDtypeStruct((N, D), jnp.float32),
)


def kernel(x, edge_index, W1, b1, W2, b2):
    src = edge_index[0].reshape(NC, NS, NCHUNK, C)
    dst = edge_index[1].reshape(NC, NS, NCHUNK, C)
    zer = jnp.zeros((C, D), jnp.float32)
    zed = jnp.zeros((C, 16), jnp.float32)
    one = jnp.ones((C, 16), jnp.float32)

    h1 = _gemm1(x, W1, b1)
    p1, d1 = _pool(h1, src, dst, zer, zed, one)
    h2 = _comb_gemm(p1, d1, W2, b2)
    p2, d2 = _pool(h2, src, dst, zer, zed, one)
    return _comb(p2, d2)


# speedup vs baseline: 8.0459x; 8.0459x over previous
import functools

import jax
import jax.numpy as jnp
from jax import lax
from jax.experimental import pallas as pl
from jax.experimental.pallas import tpu as pltpu
from jax.experimental.pallas import tpu_sc as plsc

N = 10000
E = 320000
D = 128
NC = 2
NS = 16
EPW = E // (NC * NS)
C = 125
NCHUNK = EPW // C
RPT = N // NS
RCH = 5

_mesh = plsc.VectorSubcoreMesh(
    core_axis_name="c", subcore_axis_name="s", num_cores=NC, num_subcores=NS)


def _pool_body(h_hbm, src_hbm, dst_hbm, zer_hbm, zed_hbm, one_hbm,
               part_out, deg_out,
               src_v, dst_v, rows_v, ones_v, sem, agg_sh, deg_sh):
    c = lax.axis_index("c")
    s = lax.axis_index("s")

    def zbody(i, carry):
        base = s * RPT + i * C
        pltpu.sync_copy(zer_hbm, agg_sh.at[pl.ds(base, C)])
        pltpu.sync_copy(zed_hbm, deg_sh.at[pl.ds(base, C)])
        return carry
    lax.fori_loop(0, RCH, zbody, 0)

    pltpu.sync_copy(src_hbm.at[c, s], src_v)
    pltpu.sync_copy(dst_hbm.at[c, s], dst_v)
    pltpu.sync_copy(one_hbm, ones_v)
    plsc.subcore_barrier()

    def ebody(i, carry):
        pltpu.async_copy(h_hbm.at[src_v.at[i]], rows_v, sem).wait()
        pltpu.sync_copy(rows_v, agg_sh.at[dst_v.at[i]], add=True)
        pltpu.sync_copy(ones_v, deg_sh.at[dst_v.at[i]], add=True)
        return carry
    lax.fori_loop(0, NCHUNK, ebody, 0)
    plsc.subcore_barrier()

    def obody(i, carry):
        base = s * RPT + i * C
        pltpu.sync_copy(agg_sh.at[pl.ds(base, C)], rows_v)
        pltpu.sync_copy(rows_v, part_out.at[c, pl.ds(base, C)])
        pltpu.sync_copy(deg_sh.at[pl.ds(base, C)], ones_v)
        pltpu.sync_copy(ones_v, deg_out.at[c, pl.ds(base, C)])
        return carry
    lax.fori_loop(0, RCH, obody, 0)


_pool = pl.kernel(
    _pool_body,
    out_type=(jax.ShapeDtypeStruct((NC, N, D), jnp.float32),
              jax.ShapeDtypeStruct((NC, N, 16), jnp.float32)),
    mesh=_mesh,
    compiler_params=pltpu.CompilerParams(use_tc_tiling_on_sc=False),
    scratch_types=(
        pltpu.VMEM((NCHUNK, C), jnp.int32),
        pltpu.VMEM((NCHUNK, C), jnp.int32),
        pltpu.VMEM((C, D), jnp.float32),
        pltpu.VMEM((C, 16), jnp.float32),
        pltpu.SemaphoreType.DMA,
        pltpu.VMEM_SHARED((N, D), jnp.float32),
        pltpu.VMEM_SHARED((N, 16), jnp.float32),
    ),
)


RB = 1000


def _gemm1_body(x_ref, w_ref, b_ref, o_ref):
    o_ref[...] = (jnp.dot(x_ref[...], w_ref[...],
                          preferred_element_type=jnp.float32)
                  + b_ref[...][None, :])


_gemm1 = pl.pallas_call(
    _gemm1_body,
    grid=(N // RB,),
    in_specs=[
        pl.BlockSpec((RB, D), lambda i: (i, 0)),
        pl.BlockSpec((D, D), lambda i: (0, 0)),
        pl.BlockSpec((D,), lambda i: (0,)),
    ],
    out_specs=pl.BlockSpec((RB, D), lambda i: (i, 0)),
    out_shape=jax.ShapeDtypeStruct((N, D), jnp.float32),
)


def _norm(p_ref, d_ref):
    p = p_ref[0] + p_ref[1]
    deg = jnp.sum(d_ref[0] + d_ref[1], axis=1, keepdims=True) / 16.0
    return p / jnp.maximum(deg, 1.0)


def _comb_gemm_body(p_ref, d_ref, w_ref, b_ref, o_ref):
    pooled = _norm(p_ref, d_ref)
    o_ref[...] = (jnp.dot(pooled, w_ref[...],
                          preferred_element_type=jnp.float32)
                  + b_ref[...][None, :])


_comb_gemm = pl.pallas_call(
    _comb_gemm_body,
    grid=(N // RB,),
    in_specs=[
        pl.BlockSpec((NC, RB, D), lambda i: (0, i, 0)),
        pl.BlockSpec((NC, RB, 16), lambda i: (0, i, 0)),
        pl.BlockSpec((D, D), lambda i: (0, 0)),
        pl.BlockSpec((D,), lambda i: (0,)),
    ],
    out_specs=pl.BlockSpec((RB, D), lambda i: (i, 0)),
    out_shape=jax.ShapeDtypeStruct((N, D), jnp.float32),
)


def _comb_body(p_ref, d_ref, o_ref):
    o_ref[...] = _norm(p_ref, d_ref)


_comb = pl.pallas_call(
    _comb_body,
    grid=(N // RB,),
    in_specs=[
        pl.BlockSpec((NC, RB, D), lambda i: (0, i, 0)),
        pl.BlockSpec((NC, RB, 16), lambda i: (0, i, 0)),
    ],
    out_specs=pl.BlockSpec((RB, D), lambda i: (i, 0)),
    out_shape=jax.ShapeDtypeStruct((N, D), jnp.float32),
)


def kernel(x, edge_index, W1, b1, W2, b2):
    src = edge_index[0].reshape(NC, NS, NCHUNK, C)
    dst = edge_index[1].reshape(NC, NS, NCHUNK, C)
    zer = jnp.zeros((C, D), jnp.float32)
    zed = jnp.zeros((C, 16), jnp.float32)
    one = jnp.ones((C, 16), jnp.float32)

    h1 = _gemm1(x, W1, b1)
    p1, d1 = _pool(h1, src, dst, zer, zed, one)
    h2 = _comb_gemm(p1, d1, W2, b2)
    p2, d2 = _pool(h2, src, dst, zer, zed, one)
    return _comb(p2, d2)
